# baseline (speedup 1.0000x reference)
import functools

import jax
import jax.numpy as jnp
from jax import lax
from jax.experimental import pallas as pl
from jax.experimental.pallas import tpu as pltpu
from jax.experimental.pallas import tpu_sc as plsc

S = 7
B = 2
C = 20
CH = B * 5 + C
BS = 8192
N_CELLS = BS * S * S
NC = 2
NS = 16
NW = NC * NS
L = 16
CPW = N_CELLS // NW
CHUNK = 784
NCHUNK = CPW // CHUNK
GROUPS = CHUNK // L
CW = CHUNK * CH
TOT = N_CELLS * CH
LN2 = 0.6931471805599453
LAMBDA_COORD = 5.0
LAMBDA_NOOBJ = 0.5


def _fsqrt(x):
    b = plsc.bitcast(x, jnp.int32)
    r = plsc.bitcast(0x5F375A86 - (b >> 1), jnp.float32)
    r = r * (1.5 - 0.5 * x * r * r)
    r = r * (1.5 - 0.5 * x * r * r)
    r = r * (1.5 - 0.5 * x * r * r)
    return x * r


def _flog(x):
    b = plsc.bitcast(x, jnp.int32)
    e = ((b >> 23) - 127).astype(jnp.float32)
    m = plsc.bitcast((b & 0x007FFFFF) | 0x3F800000, jnp.float32)
    t = (m - 1.0) / (m + 1.0)
    t2 = t * t
    p = 2.0 * t * (1.0 + t2 * (1.0 / 3.0 + t2 * (0.2 + t2 * (1.0 / 7.0 + t2 * (1.0 / 9.0)))))
    return e * LN2 + p


def _iou(bx, by, bw, bh, cx, cy, cw, ch):
    b1x1 = bx - bw / 2
    b1y1 = by - bh / 2
    b1x2 = bx + bw / 2
    b1y2 = by + bh / 2
    b2x1 = cx - cw / 2
    b2y1 = cy - ch / 2
    b2x2 = cx + cw / 2
    b2y2 = cy + ch / 2
    ix1 = jnp.maximum(b1x1, b2x1)
    iy1 = jnp.maximum(b1y1, b2y1)
    ix2 = jnp.minimum(b1x2, b2x2)
    iy2 = jnp.minimum(b1y2, b2y2)
    inter = jnp.maximum(ix2 - ix1, 0.0) * jnp.maximum(iy2 - iy1, 0.0)
    a1 = jnp.abs((b1x2 - b1x1) * (b1y2 - b1y1))
    a2 = jnp.abs((b2x2 - b2x1) * (b2y2 - b2y1))
    return inter / (a1 + a2 - inter + 1e-6)


def _group(pbuf, gbuf, i30, gi, accs):
    idx0 = i30 + gi * (CH * L)

    def P(c):
        return plsc.load_gather(pbuf, [idx0 + c])

    def G(c):
        return plsc.load_gather(gbuf, [idx0 + c])

    cnt, a_xy, a_wh, a_oc, a_pc2, a_pc2o, a_cell, a_nll = accs

    p0, p1, p2, p3, p4 = P(0), P(1), P(2), P(3), P(4)
    p5, p6, p7, p8, p9 = P(5), P(6), P(7), P(8), P(9)
    g0, g1, g2, g3, g4 = G(0), G(1), G(2), G(3), G(4)
    g5, g6, g7, g8 = G(5), G(6), G(7), G(8)

    iou0 = _iou(p0, p1, p2, p3, g0, g1, g2, g3)
    iou1 = _iou(p5, p6, p7, p8, g5, g6, g7, g8)
    pick1 = iou1 > iou0
    src0 = g4 > 0.0
    o0 = jnp.where(jnp.logical_and(jnp.logical_not(pick1), src0), 1.0, 0.0)
    o1 = jnp.where(jnp.logical_and(pick1, src0), 1.0, 0.0)

    def sq(v):
        return v * v

    xy = o0 * (sq(p0 - g0) + sq(p1 - g1)) + o1 * (sq(p5 - g5) + sq(p6 - g6))

    cp2 = jnp.maximum(p2, 1e-6)
    cp3 = jnp.maximum(p3, 1e-6)
    cp7 = jnp.maximum(p7, 1e-6)
    cp8 = jnp.maximum(p8, 1e-6)
    cg2 = jnp.maximum(g2, 1e-6)
    cg3 = jnp.maximum(g3, 1e-6)
    cg7 = jnp.maximum(g7, 1e-6)
    cg8 = jnp.maximum(g8, 1e-6)
    wh = o0 * (cp2 + cg2 - 2.0 * _fsqrt(cp2 * cg2) +
               cp3 + cg3 - 2.0 * _fsqrt(cp3 * cg3)) + \
         o1 * (cp7 + cg7 - 2.0 * _fsqrt(cp7 * cg7) +
               cp8 + cg8 - 2.0 * _fsqrt(cp8 * cg8))

    oc = o0 * sq(p4 - g4) + o1 * sq(p9 - g5)
    pc2 = p4 * p4 + p9 * p9
    pc2o = o0 * p4 * p4 + o1 * p9 * p9
    cellf = jnp.where((g4 + g5) > 0.0, 1.0, 0.0)

    cnt = cnt + (o0 + o1)
    a_xy = a_xy + xy
    a_wh = a_wh + wh
    a_oc = a_oc + oc
    a_pc2 = a_pc2 + pc2
    a_pc2o = a_pc2o + pc2o
    a_cell = a_cell + cellf

    pc = [P(c) for c in range(10, CH)]
    m = pc[0]
    for k in range(1, C):
        m = jnp.maximum(m, pc[k])
    ssum = lax.exp(pc[0] - m)
    for k in range(1, C):
        ssum = ssum + lax.exp(pc[k] - m)
    lse = _flog(ssum) + m

    bg = G(10)
    bi = jnp.zeros((L,), jnp.int32)
    for c in range(11, CH):
        gc = G(c)
        cond = gc > bg
        bg = jnp.where(cond, gc, bg)
        bi = jnp.where(cond, c - 10, bi)
    ptgt = plsc.load_gather(pbuf, [idx0 + 10 + bi])
    a_nll = a_nll + cellf * (lse - ptgt)

    return (cnt, a_xy, a_wh, a_oc, a_pc2, a_pc2o, a_cell, a_nll)


def _sc_body(pred_hbm, gt_hbm, out_hbm,
             pbuf0, gbuf0, pbuf1, gbuf1, obuf,
             sp0, sg0, sp1, sg1):
    wid = lax.axis_index("s") * NC + lax.axis_index("c")
    base = wid * (CPW * CH)
    i30 = lax.iota(jnp.int32, L) * CH

    def start(ci, pbuf, gbuf, semp, semg):
        off = base + ci * CW
        pltpu.async_copy(pred_hbm.at[pl.ds(off, CW)], pbuf, semp)
        pltpu.async_copy(gt_hbm.at[pl.ds(off, CW)], gbuf, semg)

    def wait(pbuf, gbuf, semp, semg):
        pltpu.make_async_copy(pred_hbm.at[pl.ds(0, CW)], pbuf, semp).wait()
        pltpu.make_async_copy(gt_hbm.at[pl.ds(0, CW)], gbuf, semg).wait()

    def compute(pbuf, gbuf, accs):
        def gb(gi, a):
            return _group(pbuf, gbuf, i30, gi, a)
        return lax.fori_loop(0, GROUPS, gb, accs, unroll=7)

    start(0, pbuf0, gbuf0, sp0, sg0)

    def body2(i, accs):
        c0 = 2 * i
        wait(pbuf0, gbuf0, sp0, sg0)
        start(c0 + 1, pbuf1, gbuf1, sp1, sg1)
        accs = compute(pbuf0, gbuf0, accs)
        wait(pbuf1, gbuf1, sp1, sg1)

        @pl.when(c0 + 2 < NCHUNK)
        def _():
            start(c0 + 2, pbuf0, gbuf0, sp0, sg0)

        return compute(pbuf1, gbuf1, accs)

    z = jnp.zeros((L,), jnp.float32)
    accs = lax.fori_loop(0, NCHUNK // 2, body2, (z,) * 8)
    for k in range(8):
        obuf[pl.ds(k * L, L)] = accs[k]
    pltpu.sync_copy(obuf, out_hbm.at[wid])


_sc_loss = functools.partial(
    pl.kernel,
    out_type=jax.ShapeDtypeStruct((NW, 8 * L), jnp.float32),
    mesh=plsc.VectorSubcoreMesh(
        core_axis_name="c", subcore_axis_name="s",
        num_cores=NC, num_subcores=NS),
    compiler_params=pltpu.CompilerParams(
        use_tc_tiling_on_sc=False, needs_layout_passes=False),
    scratch_types=[
        pltpu.VMEM((CW,), jnp.float32),
        pltpu.VMEM((CW,), jnp.float32),
        pltpu.VMEM((CW,), jnp.float32),
        pltpu.VMEM((CW,), jnp.float32),
        pltpu.VMEM((8 * L,), jnp.float32),
        pltpu.SemaphoreType.DMA,
        pltpu.SemaphoreType.DMA,
        pltpu.SemaphoreType.DMA,
        pltpu.SemaphoreType.DMA,
    ],
)(_sc_body)


def _fin_body(x_ref, o_ref):
    x = x_ref[...]
    s = [jnp.sum(x[:, k * L:(k + 1) * L]) for k in range(8)]
    cnt_obj, s_xy, s_wh, s_oc, s_pc2, s_pc2o, s_cell, s_nll = s
    cnt_noobj = float(N_CELLS * B) - cnt_obj
    xy_loss = s_xy / (2.0 * cnt_obj)
    wh_loss = s_wh / (2.0 * cnt_obj)
    loc_loss = LAMBDA_COORD * (xy_loss + wh_loss)
    conf_loss = s_oc / cnt_obj + LAMBDA_NOOBJ * (s_pc2 - s_pc2o) / cnt_noobj
    class_loss = s_nll / s_cell
    o_ref[0, 0] = (loc_loss + conf_loss + class_loss) / float(BS)


_finish = pl.pallas_call(
    _fin_body,
    out_shape=jax.ShapeDtypeStruct((1, 1), jnp.float32),
    out_specs=pl.BlockSpec(memory_space=pltpu.SMEM),
)


@jax.jit
def _run(pred, gt):
    p = pred.reshape(TOT)
    g = gt.reshape(TOT)
    partials = _sc_loss(p, g)
    return _finish(partials)[0, 0]


def kernel(pred, gt):
    return _run(pred, gt)


# speedup vs baseline: 1.6269x; 1.6269x over previous
import functools

import jax
import jax.numpy as jnp
from jax import lax
from jax.experimental import pallas as pl
from jax.experimental.pallas import tpu as pltpu
from jax.experimental.pallas import tpu_sc as plsc

S = 7
B = 2
C = 20
CH = B * 5 + C
BS = 8192
N_CELLS = BS * S * S
NC = 2
NS = 16
NW = NC * NS
L = 16
CPW = N_CELLS // NW
CHUNK = 784
NCHUNK = CPW // CHUNK
GROUPS = CHUNK // L
CW = CHUNK * CH
TOT = N_CELLS * CH
LN2 = 0.6931471805599453
LAMBDA_COORD = 5.0
LAMBDA_NOOBJ = 0.5


def _fsqrt(x):
    b = plsc.bitcast(x, jnp.int32)
    r = plsc.bitcast(0x5F375A86 - (b >> 1), jnp.float32)
    r = r * (1.5 - 0.5 * x * r * r)
    r = r * (1.5 - 0.5 * x * r * r)
    r = r * (1.5 - 0.5 * x * r * r)
    return x * r


def _flog(x):
    b = plsc.bitcast(x, jnp.int32)
    e = ((b >> 23) - 127).astype(jnp.float32)
    m = plsc.bitcast((b & 0x007FFFFF) | 0x3F800000, jnp.float32)
    t = (m - 1.0) / (m + 1.0)
    t2 = t * t
    p = 2.0 * t * (1.0 + t2 * (1.0 / 3.0 + t2 * (0.2 + t2 * (1.0 / 7.0 + t2 * (1.0 / 9.0)))))
    return e * LN2 + p


def _iou(bx, by, bw, bh, cx, cy, cw, ch):
    b1x1 = bx - bw / 2
    b1y1 = by - bh / 2
    b1x2 = bx + bw / 2
    b1y2 = by + bh / 2
    b2x1 = cx - cw / 2
    b2y1 = cy - ch / 2
    b2x2 = cx + cw / 2
    b2y2 = cy + ch / 2
    ix1 = jnp.maximum(b1x1, b2x1)
    iy1 = jnp.maximum(b1y1, b2y1)
    ix2 = jnp.minimum(b1x2, b2x2)
    iy2 = jnp.minimum(b1y2, b2y2)
    inter = jnp.maximum(ix2 - ix1, 0.0) * jnp.maximum(iy2 - iy1, 0.0)
    a1 = jnp.abs((b1x2 - b1x1) * (b1y2 - b1y1))
    a2 = jnp.abs((b2x2 - b2x1) * (b2y2 - b2y1))
    return inter / (a1 + a2 - inter + 1e-6)


def _group(pbuf, gbuf, i30, gi, accs):
    idx0 = i30 + gi * (CH * L)

    def P(c):
        return plsc.load_gather(pbuf, [idx0 + c])

    def G(c):
        return plsc.load_gather(gbuf, [idx0 + c])

    cnt, a_xy, a_wh, a_oc, a_pc2, a_pc2o, a_cell, a_nll = accs

    p0, p1, p2, p3, p4 = P(0), P(1), P(2), P(3), P(4)
    p5, p6, p7, p8, p9 = P(5), P(6), P(7), P(8), P(9)
    g0, g1, g2, g3, g4 = G(0), G(1), G(2), G(3), G(4)
    g5, g6, g7, g8 = G(5), G(6), G(7), G(8)

    iou0 = _iou(p0, p1, p2, p3, g0, g1, g2, g3)
    iou1 = _iou(p5, p6, p7, p8, g5, g6, g7, g8)
    pick1 = iou1 > iou0
    src0 = g4 > 0.0
    o0 = jnp.where(jnp.logical_and(jnp.logical_not(pick1), src0), 1.0, 0.0)
    o1 = jnp.where(jnp.logical_and(pick1, src0), 1.0, 0.0)

    def sq(v):
        return v * v

    xy = o0 * (sq(p0 - g0) + sq(p1 - g1)) + o1 * (sq(p5 - g5) + sq(p6 - g6))

    cp2 = jnp.maximum(p2, 1e-6)
    cp3 = jnp.maximum(p3, 1e-6)
    cp7 = jnp.maximum(p7, 1e-6)
    cp8 = jnp.maximum(p8, 1e-6)
    cg2 = jnp.maximum(g2, 1e-6)
    cg3 = jnp.maximum(g3, 1e-6)
    cg7 = jnp.maximum(g7, 1e-6)
    cg8 = jnp.maximum(g8, 1e-6)
    wh = o0 * (cp2 + cg2 - 2.0 * _fsqrt(cp2 * cg2) +
               cp3 + cg3 - 2.0 * _fsqrt(cp3 * cg3)) + \
         o1 * (cp7 + cg7 - 2.0 * _fsqrt(cp7 * cg7) +
               cp8 + cg8 - 2.0 * _fsqrt(cp8 * cg8))

    oc = o0 * sq(p4 - g4) + o1 * sq(p9 - g5)
    pc2 = p4 * p4 + p9 * p9
    pc2o = o0 * p4 * p4 + o1 * p9 * p9
    cellf = jnp.where((g4 + g5) > 0.0, 1.0, 0.0)

    cnt = cnt + (o0 + o1)
    a_xy = a_xy + xy
    a_wh = a_wh + wh
    a_oc = a_oc + oc
    a_pc2 = a_pc2 + pc2
    a_pc2o = a_pc2o + pc2o
    a_cell = a_cell + cellf

    pc = [P(c) for c in range(10, CH)]
    m = pc[0]
    for k in range(1, C):
        m = jnp.maximum(m, pc[k])
    ssum = lax.exp(pc[0] - m)
    for k in range(1, C):
        ssum = ssum + lax.exp(pc[k] - m)
    lse = _flog(ssum) + m

    bg = G(10)
    bi = jnp.zeros((L,), jnp.int32)
    for c in range(11, CH):
        gc = G(c)
        cond = gc > bg
        bg = jnp.where(cond, gc, bg)
        bi = jnp.where(cond, c - 10, bi)
    ptgt = plsc.load_gather(pbuf, [idx0 + 10 + bi])
    a_nll = a_nll + cellf * (lse - ptgt)

    return (cnt, a_xy, a_wh, a_oc, a_pc2, a_pc2o, a_cell, a_nll)


def _sc_body(pred_hbm, gt_hbm, out_hbm,
             pbuf0, gbuf0, pbuf1, gbuf1, obuf,
             sp0, sg0, sp1, sg1):
    wid = lax.axis_index("s") * NC + lax.axis_index("c")
    base = wid * (CPW * CH)
    i30 = lax.iota(jnp.int32, L) * CH

    def start(ci, pbuf, gbuf, semp, semg):
        off = base + ci * CW
        pltpu.async_copy(pred_hbm.at[pl.ds(off, CW)], pbuf, semp)
        pltpu.async_copy(gt_hbm.at[pl.ds(off, CW)], gbuf, semg)

    def wait(pbuf, gbuf, semp, semg):
        pltpu.make_async_copy(pred_hbm.at[pl.ds(0, CW)], pbuf, semp).wait()
        pltpu.make_async_copy(gt_hbm.at[pl.ds(0, CW)], gbuf, semg).wait()

    def compute(pbuf, gbuf, accs):
        def gb(gi, a):
            return _group(pbuf, gbuf, i30, gi, a)
        return lax.fori_loop(0, GROUPS, gb, accs, unroll=2)

    start(0, pbuf0, gbuf0, sp0, sg0)

    def body2(i, accs):
        c0 = 2 * i
        wait(pbuf0, gbuf0, sp0, sg0)
        start(c0 + 1, pbuf1, gbuf1, sp1, sg1)
        accs = compute(pbuf0, gbuf0, accs)
        wait(pbuf1, gbuf1, sp1, sg1)

        @pl.when(c0 + 2 < NCHUNK)
        def _():
            start(c0 + 2, pbuf0, gbuf0, sp0, sg0)

        return compute(pbuf1, gbuf1, accs)

    z = jnp.zeros((L,), jnp.float32)
    accs = lax.fori_loop(0, NCHUNK // 2, body2, (z,) * 8)
    for k in range(8):
        obuf[pl.ds(k * L, L)] = accs[k]
    pltpu.sync_copy(obuf, out_hbm.at[wid])


_sc_loss = functools.partial(
    pl.kernel,
    out_type=jax.ShapeDtypeStruct((NW, 8 * L), jnp.float32),
    mesh=plsc.VectorSubcoreMesh(
        core_axis_name="c", subcore_axis_name="s",
        num_cores=NC, num_subcores=NS),
    compiler_params=pltpu.CompilerParams(
        use_tc_tiling_on_sc=False, needs_layout_passes=False),
    scratch_types=[
        pltpu.VMEM((CW,), jnp.float32),
        pltpu.VMEM((CW,), jnp.float32),
        pltpu.VMEM((CW,), jnp.float32),
        pltpu.VMEM((CW,), jnp.float32),
        pltpu.VMEM((8 * L,), jnp.float32),
        pltpu.SemaphoreType.DMA,
        pltpu.SemaphoreType.DMA,
        pltpu.SemaphoreType.DMA,
        pltpu.SemaphoreType.DMA,
    ],
)(_sc_body)


def _fin_body(x_ref, o_ref):
    x = x_ref[...]
    s = [jnp.sum(x[:, k * L:(k + 1) * L]) for k in range(8)]
    cnt_obj, s_xy, s_wh, s_oc, s_pc2, s_pc2o, s_cell, s_nll = s
    cnt_noobj = float(N_CELLS * B) - cnt_obj
    xy_loss = s_xy / (2.0 * cnt_obj)
    wh_loss = s_wh / (2.0 * cnt_obj)
    loc_loss = LAMBDA_COORD * (xy_loss + wh_loss)
    conf_loss = s_oc / cnt_obj + LAMBDA_NOOBJ * (s_pc2 - s_pc2o) / cnt_noobj
    class_loss = s_nll / s_cell
    o_ref[0, 0] = (loc_loss + conf_loss + class_loss) / float(BS)


_finish = pl.pallas_call(
    _fin_body,
    out_shape=jax.ShapeDtypeStruct((1, 1), jnp.float32),
    out_specs=pl.BlockSpec(memory_space=pltpu.SMEM),
)


@jax.jit
def _run(pred, gt):
    p = pred.reshape(TOT)
    g = gt.reshape(TOT)
    partials = _sc_loss(p, g)
    return _finish(partials)[0, 0]


def kernel(pred, gt):
    return _run(pred, gt)


# speedup vs baseline: 1.6302x; 1.0020x over previous
import functools

import jax
import jax.numpy as jnp
from jax import lax
from jax.experimental import pallas as pl
from jax.experimental.pallas import tpu as pltpu
from jax.experimental.pallas import tpu_sc as plsc

S = 7
B = 2
C = 20
CH = B * 5 + C
BS = 8192
N_CELLS = BS * S * S
NC = 2
NS = 16
NW = NC * NS
L = 16
CPW = N_CELLS // NW
CHUNK = 784
NCHUNK = CPW // CHUNK
GROUPS = CHUNK // L
CW = CHUNK * CH
TOT = N_CELLS * CH
LN2 = 0.6931471805599453
LAMBDA_COORD = 5.0
LAMBDA_NOOBJ = 0.5


def _fsqrt(x):
    b = plsc.bitcast(x, jnp.int32)
    r = plsc.bitcast(0x5F375A86 - (b >> 1), jnp.float32)
    r = r * (1.5 - 0.5 * x * r * r)
    r = r * (1.5 - 0.5 * x * r * r)
    r = r * (1.5 - 0.5 * x * r * r)
    return x * r


def _flog(x):
    b = plsc.bitcast(x, jnp.int32)
    e = ((b >> 23) - 127).astype(jnp.float32)
    m = plsc.bitcast((b & 0x007FFFFF) | 0x3F800000, jnp.float32)
    t = (m - 1.0) / (m + 1.0)
    t2 = t * t
    p = 2.0 * t * (1.0 + t2 * (1.0 / 3.0 + t2 * (0.2 + t2 * (1.0 / 7.0 + t2 * (1.0 / 9.0)))))
    return e * LN2 + p


def _iou(bx, by, bw, bh, cx, cy, cw, ch):
    b1x1 = bx - bw / 2
    b1y1 = by - bh / 2
    b1x2 = bx + bw / 2
    b1y2 = by + bh / 2
    b2x1 = cx - cw / 2
    b2y1 = cy - ch / 2
    b2x2 = cx + cw / 2
    b2y2 = cy + ch / 2
    ix1 = jnp.maximum(b1x1, b2x1)
    iy1 = jnp.maximum(b1y1, b2y1)
    ix2 = jnp.minimum(b1x2, b2x2)
    iy2 = jnp.minimum(b1y2, b2y2)
    inter = jnp.maximum(ix2 - ix1, 0.0) * jnp.maximum(iy2 - iy1, 0.0)
    a1 = jnp.abs((b1x2 - b1x1) * (b1y2 - b1y1))
    a2 = jnp.abs((b2x2 - b2x1) * (b2y2 - b2y1))
    return inter / (a1 + a2 - inter + 1e-6)


def _group(pbuf, gbuf, i30, gi, accs):
    idx0 = i30 + gi * (CH * L)

    def P(c):
        return plsc.load_gather(pbuf, [idx0 + c])

    def G(c):
        return plsc.load_gather(gbuf, [idx0 + c])

    cnt, a_xy, a_wh, a_oc, a_pc2, a_pc2o, a_cell, a_nll = accs

    p0, p1, p2, p3, p4 = P(0), P(1), P(2), P(3), P(4)
    p5, p6, p7, p8, p9 = P(5), P(6), P(7), P(8), P(9)
    g0, g1, g2, g3, g4 = G(0), G(1), G(2), G(3), G(4)
    g5, g6, g7, g8 = G(5), G(6), G(7), G(8)

    iou0 = _iou(p0, p1, p2, p3, g0, g1, g2, g3)
    iou1 = _iou(p5, p6, p7, p8, g5, g6, g7, g8)
    pick1 = iou1 > iou0
    src0 = g4 > 0.0
    o0 = jnp.where(jnp.logical_and(jnp.logical_not(pick1), src0), 1.0, 0.0)
    o1 = jnp.where(jnp.logical_and(pick1, src0), 1.0, 0.0)

    def sq(v):
        return v * v

    xy = o0 * (sq(p0 - g0) + sq(p1 - g1)) + o1 * (sq(p5 - g5) + sq(p6 - g6))

    cp2 = jnp.maximum(p2, 1e-6)
    cp3 = jnp.maximum(p3, 1e-6)
    cp7 = jnp.maximum(p7, 1e-6)
    cp8 = jnp.maximum(p8, 1e-6)
    cg2 = jnp.maximum(g2, 1e-6)
    cg3 = jnp.maximum(g3, 1e-6)
    cg7 = jnp.maximum(g7, 1e-6)
    cg8 = jnp.maximum(g8, 1e-6)
    wh = o0 * (cp2 + cg2 - 2.0 * _fsqrt(cp2 * cg2) +
               cp3 + cg3 - 2.0 * _fsqrt(cp3 * cg3)) + \
         o1 * (cp7 + cg7 - 2.0 * _fsqrt(cp7 * cg7) +
               cp8 + cg8 - 2.0 * _fsqrt(cp8 * cg8))

    oc = o0 * sq(p4 - g4) + o1 * sq(p9 - g5)
    pc2 = p4 * p4 + p9 * p9
    pc2o = o0 * p4 * p4 + o1 * p9 * p9
    cellf = jnp.where((g4 + g5) > 0.0, 1.0, 0.0)

    cnt = cnt + (o0 + o1)
    a_xy = a_xy + xy
    a_wh = a_wh + wh
    a_oc = a_oc + oc
    a_pc2 = a_pc2 + pc2
    a_pc2o = a_pc2o + pc2o
    a_cell = a_cell + cellf

    pc = [P(c) for c in range(10, CH)]
    m = pc[0]
    for k in range(1, C):
        m = jnp.maximum(m, pc[k])
    ssum = lax.exp(pc[0] - m)
    for k in range(1, C):
        ssum = ssum + lax.exp(pc[k] - m)
    lse = _flog(ssum) + m

    bg = G(10)
    bi = jnp.zeros((L,), jnp.int32)
    for c in range(11, CH):
        gc = G(c)
        cond = gc > bg
        bg = jnp.where(cond, gc, bg)
        bi = jnp.where(cond, c - 10, bi)
    ptgt = plsc.load_gather(pbuf, [idx0 + 10 + bi])
    a_nll = a_nll + cellf * (lse - ptgt)

    return (cnt, a_xy, a_wh, a_oc, a_pc2, a_pc2o, a_cell, a_nll)


def _sc_body(pred_hbm, gt_hbm, out_hbm,
             pbuf0, gbuf0, pbuf1, gbuf1, obuf,
             sp0, sg0, sp1, sg1):
    wid = lax.axis_index("s") * NC + lax.axis_index("c")
    base = wid * (CPW * CH)
    i30 = lax.iota(jnp.int32, L) * CH

    def start(ci, pbuf, gbuf, semp, semg):
        off = base + ci * CW
        pltpu.async_copy(pred_hbm.at[pl.ds(off, CW)], pbuf, semp)
        pltpu.async_copy(gt_hbm.at[pl.ds(off, CW)], gbuf, semg)

    def wait(pbuf, gbuf, semp, semg):
        pltpu.make_async_copy(pred_hbm.at[pl.ds(0, CW)], pbuf, semp).wait()
        pltpu.make_async_copy(gt_hbm.at[pl.ds(0, CW)], gbuf, semg).wait()

    def compute(pbuf, gbuf, accs):
        def gb(gi, a):
            return _group(pbuf, gbuf, i30, gi, a)
        return lax.fori_loop(0, GROUPS, gb, accs, unroll=3)

    start(0, pbuf0, gbuf0, sp0, sg0)

    def body2(i, accs):
        c0 = 2 * i
        wait(pbuf0, gbuf0, sp0, sg0)
        start(c0 + 1, pbuf1, gbuf1, sp1, sg1)
        accs = compute(pbuf0, gbuf0, accs)
        wait(pbuf1, gbuf1, sp1, sg1)

        @pl.when(c0 + 2 < NCHUNK)
        def _():
            start(c0 + 2, pbuf0, gbuf0, sp0, sg0)

        return compute(pbuf1, gbuf1, accs)

    z = jnp.zeros((L,), jnp.float32)
    accs = lax.fori_loop(0, NCHUNK // 2, body2, (z,) * 8)
    for k in range(8):
        obuf[pl.ds(k * L, L)] = accs[k]
    pltpu.sync_copy(obuf, out_hbm.at[wid])


_sc_loss = functools.partial(
    pl.kernel,
    out_type=jax.ShapeDtypeStruct((NW, 8 * L), jnp.float32),
    mesh=plsc.VectorSubcoreMesh(
        core_axis_name="c", subcore_axis_name="s",
        num_cores=NC, num_subcores=NS),
    compiler_params=pltpu.CompilerParams(
        use_tc_tiling_on_sc=False, needs_layout_passes=False),
    scratch_types=[
        pltpu.VMEM((CW,), jnp.float32),
        pltpu.VMEM((CW,), jnp.float32),
        pltpu.VMEM((CW,), jnp.float32),
        pltpu.VMEM((CW,), jnp.float32),
        pltpu.VMEM((8 * L,), jnp.float32),
        pltpu.SemaphoreType.DMA,
        pltpu.SemaphoreType.DMA,
        pltpu.SemaphoreType.DMA,
        pltpu.SemaphoreType.DMA,
    ],
)(_sc_body)


def _fin_body(x_ref, o_ref):
    x = x_ref[...]
    s = [jnp.sum(x[:, k * L:(k + 1) * L]) for k in range(8)]
    cnt_obj, s_xy, s_wh, s_oc, s_pc2, s_pc2o, s_cell, s_nll = s
    cnt_noobj = float(N_CELLS * B) - cnt_obj
    xy_loss = s_xy / (2.0 * cnt_obj)
    wh_loss = s_wh / (2.0 * cnt_obj)
    loc_loss = LAMBDA_COORD * (xy_loss + wh_loss)
    conf_loss = s_oc / cnt_obj + LAMBDA_NOOBJ * (s_pc2 - s_pc2o) / cnt_noobj
    class_loss = s_nll / s_cell
    o_ref[0, 0] = (loc_loss + conf_loss + class_loss) / float(BS)


_finish = pl.pallas_call(
    _fin_body,
    out_shape=jax.ShapeDtypeStruct((1, 1), jnp.float32),
    out_specs=pl.BlockSpec(memory_space=pltpu.SMEM),
)


@jax.jit
def _run(pred, gt):
    p = pred.reshape(TOT)
    g = gt.reshape(TOT)
    partials = _sc_loss(p, g)
    return _finish(partials)[0, 0]


def kernel(pred, gt):
    return _run(pred, gt)
